# baseline (speedup 1.0000x reference)
import jax
import jax.numpy as jnp
from jax import lax
from jax.experimental import pallas as pl
from jax.experimental.pallas import tpu as pltpu
from jax.experimental.pallas import tpu_sc as plsc

NUM_SEQ = 65536
NUM_LABEL = 32
BLANK = 0

_INFO = plsc.get_sparse_core_info()
NC = _INFO.num_cores
NS = _INFO.num_subcores
LANES = _INFO.num_lanes
NW = NC * NS
ROWS_W = NUM_SEQ // NW
WORDS_W = ROWS_W * NUM_LABEL
BLOCKS_W = ROWS_W // LANES


def _tree(op, xs):
    while len(xs) > 1:
        xs = [op(xs[a], xs[a + 1]) for a in range(0, len(xs), 2)]
    return xs[0]


def _argmax16(vals, labels):
    rmax = _tree(jnp.maximum, list(vals))
    big = jnp.full((16,), NUM_LABEL, jnp.int32)
    cands = [jnp.where(vals[j] == rmax, labels[j], big)
             for j in range(len(vals))]
    return rmax, _tree(jnp.minimum, cands)


def _body(em_hbm, idx_hbm, keep_hbm, em_buf, idx_buf, keep_buf, brow):
    c = lax.axis_index("c")
    s = lax.axis_index("s")
    wid = s * NC + c
    base_row = wid * ROWS_W

    pltpu.sync_copy(em_hbm.at[pl.ds(base_row * NUM_LABEL, WORDS_W)], em_buf)

    prev_off = jnp.where(wid == 0, 0, (base_row - 1) * NUM_LABEL)
    pltpu.sync_copy(em_hbm.at[pl.ds(prev_off, NUM_LABEL)], brow)

    lane = lax.iota(jnp.int32, 16)
    v0 = brow[pl.ds(0, 16)]
    v1 = brow[pl.ds(16, 16)]
    rmax = jnp.maximum(jnp.max(v0), jnp.max(v1))
    big = jnp.int32(NUM_LABEL)
    cand0 = jnp.where(v0 == rmax, lane, big)
    cand1 = jnp.where(v1 == rmax, lane + 16, big)
    halo_idx = jnp.minimum(jnp.min(cand0), jnp.min(cand1))
    halo_idx = jnp.where(wid == 0, jnp.int32(-1), halo_idx)
    plsc.store_scatter(
        idx_buf,
        [jnp.full((16,), 7, jnp.int32)],
        jnp.broadcast_to(halo_idx, (16,)),
        mask=lane == 0,
    )

    labels = [(lane + j) & (NUM_LABEL - 1) for j in range(NUM_LABEL)]
    addrs = [lane * NUM_LABEL + labels[j] for j in range(NUM_LABEL)]

    def blk(p, carry):
        base = p * (LANES * NUM_LABEL)
        vals = [plsc.load_gather(em_buf, [addrs[j] + base])
                for j in range(NUM_LABEL)]
        _, bi = _argmax16(vals, labels)
        off = p * LANES
        idx_buf[pl.ds(8 + off, 16)] = bi
        prev = plsc.load_gather(idx_buf, [lane + (7 + off)])
        keep = jnp.logical_and(bi != prev, bi != BLANK)
        keep_buf[pl.ds(off, 16)] = keep.astype(jnp.int32)
        return carry

    lax.fori_loop(0, BLOCKS_W, blk, 0)

    pltpu.sync_copy(idx_buf.at[pl.ds(8, ROWS_W)], idx_hbm.at[pl.ds(base_row, ROWS_W)])
    pltpu.sync_copy(keep_buf, keep_hbm.at[pl.ds(base_row, ROWS_W)])


@jax.jit
def _decode(em_flat):
    k = pl.kernel(
        _body,
        out_type=[
            jax.ShapeDtypeStruct((NUM_SEQ,), jnp.int32),
            jax.ShapeDtypeStruct((NUM_SEQ,), jnp.int32),
        ],
        mesh=plsc.VectorSubcoreMesh(core_axis_name="c", subcore_axis_name="s"),
        compiler_params=pltpu.CompilerParams(needs_layout_passes=False),
        scratch_types=[
            pltpu.VMEM((WORDS_W,), jnp.float32),
            pltpu.VMEM((8 + ROWS_W,), jnp.int32),
            pltpu.VMEM((ROWS_W,), jnp.int32),
            pltpu.VMEM((NUM_LABEL,), jnp.float32),
        ],
    )
    return k(em_flat)


def kernel(emission):
    idx, keep = _decode(emission.reshape(-1))
    return idx, keep.astype(jnp.bool_)
